# baseline (speedup 1.0000x reference)
import functools
import math

import jax
import jax.numpy as jnp
from jax import lax
from jax.experimental import pallas as pl
from jax.experimental.pallas import tpu as pltpu
from jax.experimental.pallas import tpu_sc as plsc

VOCAB_ROWS = 1000000
D = 64
B_TOTAL = 4096 * 200
NC, NS, L = 2, 16, 16
NW = NC * NS
CHUNK = 128
PER_W = B_TOTAL // NW
N_CHUNKS = PER_W // CHUNK
SCALE = math.sqrt(D)


def _body(x_hbm, table_hbm, out_hbm, idx_v, rows_v, sem):
    wid = lax.axis_index("s") * NC + lax.axis_index("c")
    pltpu.sync_copy(x_hbm.at[pl.ds(wid * N_CHUNKS, N_CHUNKS)], idx_v)

    @pl.loop(0, N_CHUNKS)
    def chunk_loop(t):
        pltpu.async_copy(table_hbm.at[idx_v.at[t]], rows_v, sem).wait()

        @pl.loop(0, CHUNK)
        def scale_loop(i):
            for j in range(D // L):
                sl = pl.ds(j * L, L)
                rows_v[i, sl] = rows_v[i, sl] * SCALE

        pltpu.sync_copy(
            rows_v, out_hbm.at[pl.ds((wid * N_CHUNKS + t) * CHUNK, CHUNK)]
        )


@jax.jit
def _embed(x2d, table):
    mesh = plsc.VectorSubcoreMesh(
        core_axis_name="c", subcore_axis_name="s", num_cores=NC, num_subcores=NS
    )
    run = pl.kernel(
        _body,
        out_type=jax.ShapeDtypeStruct((B_TOTAL, D), jnp.float32),
        mesh=mesh,
        scratch_types=[
            pltpu.VMEM((N_CHUNKS, CHUNK), jnp.int32),
            pltpu.VMEM((CHUNK, D), jnp.float32),
            pltpu.SemaphoreType.DMA,
        ],
        compiler_params=pltpu.CompilerParams(use_tc_tiling_on_sc=False),
    )
    return run(x2d, table)


def kernel(x, pretrained_vector):
    x2d = x.reshape(B_TOTAL // CHUNK, CHUNK).astype(jnp.int32)
    out = _embed(x2d, pretrained_vector)
    return out.reshape(x.shape + (D,))


# speedup vs baseline: 1.0954x; 1.0954x over previous
import math

import jax
import jax.numpy as jnp
from jax import lax
from jax.experimental import pallas as pl
from jax.experimental.pallas import tpu as pltpu
from jax.experimental.pallas import tpu_sc as plsc

VOCAB_ROWS = 1000000
D = 64
B_TOTAL = 4096 * 200
NC, NS, L = 2, 16, 16
NW = NC * NS
CHUNK = 128
PER_W = B_TOTAL // NW
N_CHUNKS = PER_W // CHUNK
NBUF = 4
SCALE = math.sqrt(D)


def _body(x_hbm, table_hbm, out_hbm, idx_v, in_v, out_v, gsem, ssem):
    wid = lax.axis_index("s") * NC + lax.axis_index("c")
    base_chunk = wid * N_CHUNKS
    pltpu.sync_copy(x_hbm.at[pl.ds(base_chunk, N_CHUNKS)], idx_v)

    def start_gather(t, b):
        pltpu.async_copy(table_hbm.at[idx_v.at[t]], in_v.at[b], gsem.at[b])

    def wait_gather(b):
        pltpu.make_async_copy(
            table_hbm.at[idx_v.at[0]], in_v.at[b], gsem.at[b]
        ).wait()

    def start_store(t, b):
        pltpu.async_copy(
            out_v.at[b], out_hbm.at[pl.ds((base_chunk + t) * CHUNK, CHUNK)],
            ssem.at[b],
        )

    def wait_store(b):
        pltpu.make_async_copy(
            out_v.at[b], out_hbm.at[pl.ds(0, CHUNK)], ssem.at[b]
        ).wait()

    def scale(b):
        @pl.loop(0, CHUNK, unroll=4)
        def row_loop(i):
            for j in range(D // L):
                sl = pl.ds(j * L, L)
                out_v[b, i, sl] = in_v[b, i, sl] * SCALE

    for b in range(NBUF):
        start_gather(b, b)

    for b in range(NBUF):
        wait_gather(b)
        scale(b)
        start_store(b, b)
        start_gather(b + NBUF, b)

    @pl.loop(NBUF, N_CHUNKS - NBUF, step=NBUF)
    def round_loop(t0):
        for b in range(NBUF):
            wait_gather(b)
            wait_store(b)
            scale(b)
            start_store(t0 + b, b)
            start_gather(t0 + b + NBUF, b)

    for b in range(NBUF):
        t = N_CHUNKS - NBUF + b
        wait_gather(b)
        wait_store(b)
        scale(b)
        start_store(t, b)

    for b in range(NBUF):
        wait_store(b)


@jax.jit
def _embed(x2d, table):
    mesh = plsc.VectorSubcoreMesh(
        core_axis_name="c", subcore_axis_name="s", num_cores=NC, num_subcores=NS
    )
    run = pl.kernel(
        _body,
        out_type=jax.ShapeDtypeStruct((B_TOTAL, D), jnp.float32),
        mesh=mesh,
        scratch_types=[
            pltpu.VMEM((N_CHUNKS, CHUNK), jnp.int32),
            pltpu.VMEM((NBUF, CHUNK, D), jnp.float32),
            pltpu.VMEM((NBUF, CHUNK, D), jnp.float32),
            pltpu.SemaphoreType.DMA((NBUF,)),
            pltpu.SemaphoreType.DMA((NBUF,)),
        ],
        compiler_params=pltpu.CompilerParams(use_tc_tiling_on_sc=False),
    )
    return run(x2d, table)


def kernel(x, pretrained_vector):
    x2d = x.reshape(B_TOTAL // CHUNK, CHUNK).astype(jnp.int32)
    out = _embed(x2d, pretrained_vector)
    return out.reshape(x.shape + (D,))
